# baseline (speedup 1.0000x reference)
import functools

import jax
import jax.numpy as jnp
from jax.experimental import pallas as pl
from jax.experimental.pallas import tpu as pltpu


def _mm_kernel(a_ref, w_ref, o_ref):
    o_ref[...] = jnp.dot(a_ref[...], w_ref[...],
                         preferred_element_type=jnp.float32)


def _mm(a, w, bm=1024):
    m, k = a.shape
    k2, n = w.shape
    grid = (pl.cdiv(m, bm),)
    return pl.pallas_call(
        _mm_kernel,
        grid=grid,
        in_specs=[
            pl.BlockSpec((bm, k), lambda i: (i, 0)),
            pl.BlockSpec((k2, n), lambda i: (0, 0)),
        ],
        out_specs=pl.BlockSpec((bm, n), lambda i: (i, 0)),
        out_shape=jax.ShapeDtypeStruct((m, n), jnp.float32),
    )(a, w)


def _node_stage_kernel(s_ref, mx_ref, sq_ref, cnt_ref, w2_ref, b2_ref, o_ref):
    cnt = cnt_ref[...]
    mean = s_ref[...] / cnt
    sq = sq_ref[...] / cnt
    std = jnp.sqrt(jnp.clip(sq - mean * mean, 0.0, None) + 1e-8)
    agg = jnp.concatenate([mean, mx_ref[...], std], axis=-1)
    o_ref[...] = jnp.dot(agg, w2_ref[...],
                         preferred_element_type=jnp.float32) + b2_ref[...]


def _node_stage(s, mx, sq, cnt, w2, b2, bm=2048):
    n, d = s.shape
    grid = (pl.cdiv(n, bm),)
    return pl.pallas_call(
        _node_stage_kernel,
        grid=grid,
        in_specs=[
            pl.BlockSpec((bm, d), lambda i: (i, 0)),
            pl.BlockSpec((bm, d), lambda i: (i, 0)),
            pl.BlockSpec((bm, d), lambda i: (i, 0)),
            pl.BlockSpec((bm, 1), lambda i: (i, 0)),
            pl.BlockSpec((3 * d, d), lambda i: (0, 0)),
            pl.BlockSpec((1, d), lambda i: (0, 0)),
        ],
        out_specs=pl.BlockSpec((bm, d), lambda i: (i, 0)),
        out_shape=jax.ShapeDtypeStruct((n, d), jnp.float32),
    )(s, mx, sq, cnt, w2, b2)


def _bn_res_kernel(o_ref, h_ref, mu_ref, var_ref, g_ref, bt_ref, out_ref):
    xhat = (o_ref[...] - mu_ref[...]) * jax.lax.rsqrt(var_ref[...] + 1e-5)
    out_ref[...] = jnp.maximum(xhat * g_ref[...] + bt_ref[...] + h_ref[...],
                               0.0)


def _bn_res(out, h, mu, var, gamma, beta, bm=2048):
    n, d = out.shape
    grid = (pl.cdiv(n, bm),)
    return pl.pallas_call(
        _bn_res_kernel,
        grid=grid,
        in_specs=[
            pl.BlockSpec((bm, d), lambda i: (i, 0)),
            pl.BlockSpec((bm, d), lambda i: (i, 0)),
            pl.BlockSpec((1, d), lambda i: (0, 0)),
            pl.BlockSpec((1, d), lambda i: (0, 0)),
            pl.BlockSpec((1, d), lambda i: (0, 0)),
            pl.BlockSpec((1, d), lambda i: (0, 0)),
        ],
        out_specs=pl.BlockSpec((bm, d), lambda i: (i, 0)),
        out_shape=jax.ShapeDtypeStruct((n, d), jnp.float32),
    )(out, h, mu, var, gamma, beta)


def _final_kernel(h_ref, w_ref, b_ref, o_ref):
    o_ref[...] = jnp.dot(h_ref[...], w_ref[...],
                         preferred_element_type=jnp.float32) + b_ref[...]


def _final(h, w, b, bm=2048):
    n, d = h.shape
    grid = (pl.cdiv(n, bm),)
    return pl.pallas_call(
        _final_kernel,
        grid=grid,
        in_specs=[
            pl.BlockSpec((bm, d), lambda i: (i, 0)),
            pl.BlockSpec((d, d), lambda i: (0, 0)),
            pl.BlockSpec((1, d), lambda i: (0, 0)),
        ],
        out_specs=pl.BlockSpec((bm, d), lambda i: (i, 0)),
        out_shape=jax.ShapeDtypeStruct((n, d), jnp.float32),
    )(h, w, b)


def kernel(x, edge_index, edge_attr, W1, b1, W2, b2, gamma, beta, W_out,
           b_out):
    n, d = x.shape
    src = edge_index[0].astype(jnp.int32)
    dst = edge_index[1].astype(jnp.int32)
    nlayer = W1.shape[0]

    ones = jnp.ones((src.shape[0], 1), dtype=jnp.float32)
    cnt = jax.ops.segment_sum(ones, dst, num_segments=n)
    cnt = jnp.maximum(cnt, 1.0)

    h = x
    for l in range(nlayer):
        Wa = W1[l, :d]
        Wb = W1[l, d:2 * d]
        Wc = W1[l, 2 * d:]
        ha = _mm(h, Wa)
        hb = _mm(h, Wb) + b1[l]
        ec = _mm(edge_attr, Wc, bm=4096)
        m = jax.nn.relu(ha[src] + hb[dst] + ec)
        s = jax.ops.segment_sum(m, dst, num_segments=n)
        mx = jax.ops.segment_max(m, dst, num_segments=n)
        mx = jnp.where(jnp.isfinite(mx), mx, 0.0)
        sq = jax.ops.segment_sum(m * m, dst, num_segments=n)
        out = _node_stage(s, mx, sq, cnt, W2[l], b2[l][None, :])
        mu = out.mean(axis=0, keepdims=True)
        var = out.var(axis=0, keepdims=True)
        h = _bn_res(out, h, mu, var, gamma[l][None, :], beta[l][None, :])
    return _final(h, W_out, b_out[None, :])


# speedup vs baseline: 1.6756x; 1.6756x over previous
import functools

import jax
import jax.numpy as jnp
from jax import lax
from jax.experimental import pallas as pl
from jax.experimental.pallas import tpu as pltpu
from jax.experimental.pallas import tpu_sc as plsc

N = 10000
E = 160000
D = 256
DE = 16
NB = 157
BK = 64
NBP = 176
NT = 16
EPC = 80000
EPT = 5000
EPTP = 5120
ZCH = 5168
EPC_PAD = 82688
ETOT = 2 * EPC_PAD + 64
EOUT = ETOT + 16
DUMP = ETOT
K = 64

_mesh = plsc.VectorSubcoreMesh(core_axis_name="c", subcore_axis_name="s")


def _lane(v, i):
    acc = jnp.zeros((), v.dtype)
    for q in range(16):
        acc = jnp.where(i == q, v[q], acc)
    return acc


def _vec16(vals, dtype=jnp.int32):
    lanes = lax.iota(dtype, 16) if dtype == jnp.int32 else None
    ilanes = lax.iota(jnp.int32, 16)
    v = jnp.zeros((16,), dtype)
    for q, x in enumerate(vals):
        v = jnp.where(ilanes == q, x, v)
    return v


@functools.partial(
    pl.kernel,
    mesh=_mesh,
    out_type=[
        jax.ShapeDtypeStruct((EOUT,), jnp.int32),
        jax.ShapeDtypeStruct((EOUT,), jnp.int32),
        jax.ShapeDtypeStruct((EOUT,), jnp.int32),
        jax.ShapeDtypeStruct((2 * NBP,), jnp.int32),
        jax.ShapeDtypeStruct((2 * NBP,), jnp.int32),
    ],
    scratch_types=[
        pltpu.VMEM((EPTP,), jnp.int32),
        pltpu.VMEM((EPTP,), jnp.int32),
        pltpu.VMEM((EPTP,), jnp.int32),
        pltpu.VMEM((40, 128), jnp.int32),
        pltpu.VMEM((ZCH,), jnp.int32),
        pltpu.VMEM((NBP,), jnp.int32),
        pltpu.VMEM((NT * NBP,), jnp.int32),
        pltpu.VMEM((NBP,), jnp.int32),
        pltpu.VMEM((NBP,), jnp.int32),
        pltpu.VMEM((NBP,), jnp.int32),
        pltpu.VMEM((NBP,), jnp.int32),
        pltpu.VMEM_SHARED((NT * NBP,), jnp.int32),
        pltpu.SMEM((NBP,), jnp.int32),
        pltpu.SMEM((NBP,), jnp.int32),
        pltpu.SMEM((NBP,), jnp.int32),
        pltpu.SMEM((NBP,), jnp.int32),
        pltpu.SemaphoreType.DMA,
        pltpu.SemaphoreType.DMA,
        pltpu.SemaphoreType.DMA,
    ],
)
def _bucket(src_h, dst_h, srcs_o, dsts_o, eids_o, starts_o, counts_o,
            dstv, srcv, eidv, pos2d, zerov, hv, ah, stv, ctv, totv, prev,
            hist_sh, hist, wcur, stsc, ctsc, sem0, sem1, sem2):
    c = lax.axis_index("c")
    s = lax.axis_index("s")

    z16 = jnp.zeros((16,), jnp.int32)

    def _z(i, _):
        zerov[pl.ds(pl.multiple_of(i * 16, 16), 16)] = z16
        return 0

    lax.fori_loop(0, ZCH // 16, _z, 0)

    zbase = c * EPC_PAD + s * ZCH
    pltpu.sync_copy(zerov, srcs_o.at[pl.ds(zbase, ZCH)])
    pltpu.sync_copy(zerov, dsts_o.at[pl.ds(zbase, ZCH)])
    pltpu.sync_copy(zerov, eids_o.at[pl.ds(zbase, ZCH)])

    @pl.when(jnp.logical_and(c == 1, s == NT - 1))
    def _():
        pltpu.sync_copy(zerov.at[pl.ds(0, 80)],
                        srcs_o.at[pl.ds(2 * EPC_PAD, 80)])
        pltpu.sync_copy(zerov.at[pl.ds(0, 80)],
                        dsts_o.at[pl.ds(2 * EPC_PAD, 80)])
        pltpu.sync_copy(zerov.at[pl.ds(0, 80)],
                        eids_o.at[pl.ds(2 * EPC_PAD, 80)])

    ebase = c * EPC + s * EPT
    pltpu.sync_copy(dst_h.at[pl.ds(ebase, EPT)], dstv.at[pl.ds(0, EPT)])
    pltpu.sync_copy(src_h.at[pl.ds(ebase, EPT)], srcv.at[pl.ds(0, EPT)])
    lane16 = lax.iota(jnp.int32, 16)

    def _eid(g, _):
        eidv[pl.ds(pl.multiple_of(g * 16, 16), 16)] = ebase + g * 16 + lane16
        return 0

    lax.fori_loop(0, EPTP // 16, _eid, 0)

    def _hz(b, _):
        hist[b] = 0
        return 0

    lax.fori_loop(0, NBP, _hz, 0)

    def _hb(g, _):
        dv = dstv[pl.ds(pl.multiple_of(g * 16, 16), 16)] >> 6
        for q in range(16):
            b = dv[q]
            hist[b] = hist[b] + 1
        return 0

    lax.fori_loop(0, EPT // 16, _hb, 0)
    dvt = dstv[pl.ds(4992, 16)] >> 6
    for q in range(8):
        b = dvt[q]
        hist[b] = hist[b] + 1

    for g in range(NBP // 16):
        hv[pl.ds(g * 16, 16)] = _vec16(
            [hist[g * 16 + q] for q in range(16)])
    pltpu.sync_copy(hv, hist_sh.at[pl.ds(s * NBP, NBP)])
    plsc.subcore_barrier()
    pltpu.sync_copy(hist_sh, ah)

    zi16 = jnp.zeros((16,), jnp.int32)
    for g in range(NBP // 16):
        slg = pl.ds(g * 16, 16)
        tot = zi16
        pre = zi16
        for t in range(NT):
            row = ah[pl.ds(t * NBP + g * 16, 16)]
            pre = pre + jnp.where(jnp.int32(t) < s, row, 0)
            tot = tot + row
        totv[slg] = tot
        prev[slg] = pre

    A = jnp.int32(0)
    for g in range(NBP // 16):
        tv = totv[pl.ds(g * 16, 16)]
        pv = prev[pl.ds(g * 16, 16)]
        for q in range(16):
            b = g * 16 + q
            if b >= NB:
                break
            stsc[b] = A
            ctsc[b] = tv[q]
            wcur[b] = A + pv[q]
            A = (A + tv[q] + 15) & (-16)

    @pl.when(s == 0)
    def _():
        for g in range(NBP // 16):
            stv[pl.ds(g * 16, 16)] = _vec16(
                [stsc[g * 16 + q] for q in range(16)])
            ctv[pl.ds(g * 16, 16)] = _vec16(
                [ctsc[g * 16 + q] for q in range(16)])
        pltpu.sync_copy(stv, starts_o.at[pl.ds(c * NBP, NBP)])
        pltpu.sync_copy(ctv, counts_o.at[pl.ds(c * NBP, NBP)])

    def _pf(r, _):
        for q in range(8):
            pos2d[r, pl.ds(q * 16, 16)] = jnp.full((16,), DUMP, jnp.int32)
        return 0

    lax.fori_loop(0, 40, _pf, 0)

    coff = c * EPC_PAD

    def _perm(r, _):
        for q8 in range(8):
            dv = dstv[pl.ds(pl.multiple_of(r * 128 + q8 * 16, 16), 16)] >> 6
            ps = []
            for q in range(16):
                b = dv[q]
                p = wcur[b]
                wcur[b] = p + 1
                ps.append(p + coff)
            pos2d[r, pl.ds(q8 * 16, 16)] = _vec16(ps)
        return 0

    lax.fori_loop(0, 39, _perm, 0)
    dvt2 = dstv[pl.ds(4992, 16)] >> 6
    pst = []
    for q in range(8):
        b = dvt2[q]
        p = wcur[b]
        wcur[b] = p + 1
        pst.append(p + coff)
    pos2d[39, pl.ds(0, 16)] = _vec16(
        pst + [jnp.int32(DUMP)] * 8)

    for j in range(40):
        c0 = pltpu.async_copy(
            srcv.at[pl.ds(j * 128, 128)], srcs_o.at[pos2d.at[j]], sem0)
        c1 = pltpu.async_copy(
            dstv.at[pl.ds(j * 128, 128)], dsts_o.at[pos2d.at[j]], sem1)
        c2 = pltpu.async_copy(
            eidv.at[pl.ds(j * 128, 128)], eids_o.at[pos2d.at[j]], sem2)
        c0.wait()
        c1.wait()
        c2.wait()


@functools.partial(
    pl.kernel,
    mesh=_mesh,
    out_type=[
        jax.ShapeDtypeStruct((N, D), jnp.float32),
        jax.ShapeDtypeStruct((N, D), jnp.float32),
        jax.ShapeDtypeStruct((N, D), jnp.float32),
        jax.ShapeDtypeStruct((N, 16), jnp.float32),
    ],
    scratch_types=[
        pltpu.VMEM((K,), jnp.int32),
        pltpu.VMEM((K + 16,), jnp.int32),
        pltpu.VMEM((K,), jnp.int32),
        pltpu.VMEM((K, D), jnp.float32),
        pltpu.VMEM((K, D), jnp.float32),
        pltpu.VMEM((BK, D), jnp.float32),
        pltpu.VMEM((BK + 1, D), jnp.float32),
        pltpu.VMEM((BK + 1, D), jnp.float32),
        pltpu.VMEM((BK + 1, D), jnp.float32),
        pltpu.VMEM((BK + 1, 16), jnp.float32),
        pltpu.VMEM((2 * NBP,), jnp.int32),
        pltpu.VMEM((2 * NBP,), jnp.int32),
        pltpu.SemaphoreType.DMA,
        pltpu.SemaphoreType.DMA,
    ],
)
def _edge(ha_h, hb_h, ec_h, srcs_h, dsts_h, eids_h, starts_h, counts_h,
          sum_o, sq_o, mx_o, cw_o,
          sidv, didv, eiv, har, ecr, hbl, asum, asq, amx, acw, stv, ctv,
          sem, sem2):
    c = lax.axis_index("c")
    s = lax.axis_index("s")
    wid = s * 2 + c

    pltpu.sync_copy(starts_h, stv)
    pltpu.sync_copy(counts_h, ctv)

    zf = jnp.zeros((16,), jnp.float32)
    one16 = jnp.ones((16,), jnp.float32)

    def _bucket_i(i, _0):
        bb = wid + 32 * i

        @pl.when(bb < NB)
        def _(bb=bb):
            def _za(r, _1):
                def _zj(j, _2):
                    sl = pl.ds(pl.multiple_of(j * 16, 16), 16)
                    asum[r, sl] = zf
                    asq[r, sl] = zf
                    amx[r, sl] = zf
                    return 0

                lax.fori_loop(0, D // 16, _zj, 0, unroll=4)
                acw[r, :] = zf
                return 0

            lax.fori_loop(0, BK + 1, _za, 0)

            pltpu.sync_copy(hb_h.at[pl.ds(bb * BK, BK), :], hbl)

            gb = pl.multiple_of((bb // 16) * 16, 16)
            lb = bb % 16

            def _region(reg, _1, bb=bb, gb=gb, lb=lb):
                goff = pl.multiple_of(reg * NBP + gb, 16)
                s0 = _lane(stv[pl.ds(goff, 16)], lb)
                kk = _lane(ctv[pl.ds(goff, 16)], lb)
                base = reg * EPC_PAD + s0

                def _chunk(ci, _2, base=base, kk=kk, bb=bb):
                    cb = pl.multiple_of(base + ci * K, 16)
                    pltpu.sync_copy(srcs_h.at[pl.ds(cb, K)], sidv)
                    pltpu.sync_copy(dsts_h.at[pl.ds(cb, K)],
                                    didv.at[pl.ds(0, K)])
                    pltpu.sync_copy(eids_h.at[pl.ds(cb, K)], eiv)
                    gcp = pltpu.async_copy(ha_h.at[sidv], har, sem)
                    gcp2 = pltpu.async_copy(ec_h.at[eiv], ecr, sem2)
                    gcp.wait()
                    gcp2.wait()
                    kc = jnp.minimum(kk - ci * K, K)

                    def _grp(g, _3, bb=bb, kc=kc):
                        dvec = didv[pl.ds(pl.multiple_of(g * 16, 16), 16)]
                        dvec = dvec - bb * BK
                        for q in range(16):
                            e = g * 16 + q
                            valid = e < kc
                            ld = jnp.where(valid, dvec[q], 0)
                            ldw = jnp.where(valid, dvec[q], BK)

                            def _feat(j, _4, e=e, ld=ld, ldw=ldw):
                                sl = pl.ds(pl.multiple_of(j * 16, 16), 16)
                                m = jnp.maximum(
                                    har[e, sl] + hbl[ld, sl] + ecr[e, sl],
                                    0.0)
                                plsc.addupdate(asum.at[ldw, sl], m)
                                plsc.addupdate(asq.at[ldw, sl], m * m)
                                amx[ldw, sl] = jnp.maximum(amx[ldw, sl], m)
                                return 0

                            lax.fori_loop(0, D // 16, _feat, 0, unroll=4)
                            plsc.addupdate(acw.at[ldw], one16)
                        return 0

                    lax.fori_loop(0, K // 16, _grp, 0)
                    return 0

                nch = (kk + K - 1) // K
                lax.fori_loop(0, nch, _chunk, 0)
                return 0

            lax.fori_loop(0, 2, _region, 0)

            @pl.when(bb < NB - 1)
            def _(bb=bb):
                pltpu.sync_copy(asum.at[pl.ds(0, BK), :],
                                sum_o.at[pl.ds(bb * BK, BK), :])
                pltpu.sync_copy(asq.at[pl.ds(0, BK), :],
                                sq_o.at[pl.ds(bb * BK, BK), :])
                pltpu.sync_copy(amx.at[pl.ds(0, BK), :],
                                mx_o.at[pl.ds(bb * BK, BK), :])
                pltpu.sync_copy(acw.at[pl.ds(0, BK), :],
                                cw_o.at[pl.ds(bb * BK, BK), :])

            @pl.when(bb == NB - 1)
            def _(bb=bb):
                tail = N - (NB - 1) * BK
                pltpu.sync_copy(asum.at[pl.ds(0, tail), :],
                                sum_o.at[pl.ds(bb * BK, tail), :])
                pltpu.sync_copy(asq.at[pl.ds(0, tail), :],
                                sq_o.at[pl.ds(bb * BK, tail), :])
                pltpu.sync_copy(amx.at[pl.ds(0, tail), :],
                                mx_o.at[pl.ds(bb * BK, tail), :])
                pltpu.sync_copy(acw.at[pl.ds(0, tail), :],
                                cw_o.at[pl.ds(bb * BK, tail), :])
        return 0

    lax.fori_loop(0, 5, _bucket_i, 0)


def _mm_kernel(a_ref, w_ref, o_ref):
    o_ref[...] = jnp.dot(a_ref[...], w_ref[...],
                         preferred_element_type=jnp.float32)


def _mm(a, w, bm=1024):
    m, k = a.shape
    k2, n = w.shape
    return pl.pallas_call(
        _mm_kernel,
        grid=(pl.cdiv(m, bm),),
        in_specs=[
            pl.BlockSpec((bm, k), lambda i: (i, 0)),
            pl.BlockSpec((k2, n), lambda i: (0, 0)),
        ],
        out_specs=pl.BlockSpec((bm, n), lambda i: (i, 0)),
        out_shape=jax.ShapeDtypeStruct((m, n), jnp.float32),
    )(a, w)


def _node_stage_kernel(s_ref, mx_ref, sq_ref, cw_ref, w2_ref, b2_ref, o_ref):
    cnt = jnp.maximum(cw_ref[:, :1], 1.0)
    mean = s_ref[...] / cnt
    sq = sq_ref[...] / cnt
    std = jnp.sqrt(jnp.clip(sq - mean * mean, 0.0, None) + 1e-8)
    agg = jnp.concatenate([mean, mx_ref[...], std], axis=-1)
    o_ref[...] = jnp.dot(agg, w2_ref[...],
                         preferred_element_type=jnp.float32) + b2_ref[...]


def _node_stage(s, mx, sq, cw, w2, b2, bm=2048):
    n, d = s.shape
    return pl.pallas_call(
        _node_stage_kernel,
        grid=(pl.cdiv(n, bm),),
        in_specs=[
            pl.BlockSpec((bm, d), lambda i: (i, 0)),
            pl.BlockSpec((bm, d), lambda i: (i, 0)),
            pl.BlockSpec((bm, d), lambda i: (i, 0)),
            pl.BlockSpec((bm, 16), lambda i: (i, 0)),
            pl.BlockSpec((3 * d, d), lambda i: (0, 0)),
            pl.BlockSpec((1, d), lambda i: (0, 0)),
        ],
        out_specs=pl.BlockSpec((bm, d), lambda i: (i, 0)),
        out_shape=jax.ShapeDtypeStruct((n, d), jnp.float32),
    )(s, mx, sq, cw, w2, b2)


def _bn_stats_kernel(o_ref, st_ref):
    mu = jnp.mean(o_ref[...], axis=0, keepdims=True)
    var = jnp.mean(o_ref[...] * o_ref[...], axis=0, keepdims=True) - mu * mu
    st_ref[...] = jnp.concatenate([mu, var], axis=0)


def _bn_stats(out):
    n, d = out.shape
    return pl.pallas_call(
        _bn_stats_kernel,
        out_shape=jax.ShapeDtypeStruct((2, d), jnp.float32),
    )(out)


def _bn_res_kernel(o_ref, h_ref, st_ref, g_ref, bt_ref, out_ref):
    mu = st_ref[:1]
    var = st_ref[1:2]
    xhat = (o_ref[...] - mu) * lax.rsqrt(var + 1e-5)
    out_ref[...] = jnp.maximum(xhat * g_ref[...] + bt_ref[...] + h_ref[...],
                               0.0)


def _bn_res(out, h, st, gamma, beta, bm=2048):
    n, d = out.shape
    return pl.pallas_call(
        _bn_res_kernel,
        grid=(pl.cdiv(n, bm),),
        in_specs=[
            pl.BlockSpec((bm, d), lambda i: (i, 0)),
            pl.BlockSpec((bm, d), lambda i: (i, 0)),
            pl.BlockSpec((2, d), lambda i: (0, 0)),
            pl.BlockSpec((1, d), lambda i: (0, 0)),
            pl.BlockSpec((1, d), lambda i: (0, 0)),
        ],
        out_specs=pl.BlockSpec((bm, d), lambda i: (i, 0)),
        out_shape=jax.ShapeDtypeStruct((n, d), jnp.float32),
    )(out, h, st, gamma, beta)


def _final_kernel(h_ref, w_ref, b_ref, o_ref):
    o_ref[...] = jnp.dot(h_ref[...], w_ref[...],
                         preferred_element_type=jnp.float32) + b_ref[...]


def _final(h, w, b, bm=2048):
    n, d = h.shape
    return pl.pallas_call(
        _final_kernel,
        grid=(pl.cdiv(n, bm),),
        in_specs=[
            pl.BlockSpec((bm, d), lambda i: (i, 0)),
            pl.BlockSpec((d, d), lambda i: (0, 0)),
            pl.BlockSpec((1, d), lambda i: (0, 0)),
        ],
        out_specs=pl.BlockSpec((bm, d), lambda i: (i, 0)),
        out_shape=jax.ShapeDtypeStruct((n, d), jnp.float32),
    )(h, w, b)


def kernel(x, edge_index, edge_attr, W1, b1, W2, b2, gamma, beta, W_out,
           b_out):
    n, d = x.shape
    src = edge_index[0].astype(jnp.int32)
    dst = edge_index[1].astype(jnp.int32)
    nlayer = W1.shape[0]

    srcs, dsts, eids, starts, counts = _bucket(src, dst)

    h = x
    for l in range(nlayer):
        Wa = W1[l, :d]
        Wb = W1[l, d:2 * d]
        Wc = W1[l, 2 * d:]
        ha = _mm(h, Wa)
        hb = _mm(h, Wb) + b1[l]
        ec = _mm(edge_attr, Wc, bm=4096)
        ssum, ssq, smx, cw = _edge(ha, hb, ec, srcs, dsts, eids, starts,
                                   counts)
        out = _node_stage(ssum, smx, ssq, cw, W2[l], b2[l][None, :])
        st = _bn_stats(out)
        h = _bn_res(out, h, st, gamma[l][None, :], beta[l][None, :])
    return _final(h, W_out, b_out[None, :])
